# Initial kernel scaffold; baseline (speedup 1.0000x reference)
import jax
import jax.numpy as jnp
from jax.experimental import pallas as pl


def kernel(x, weight):
    raise NotImplementedError("write your pallas kernel here")


# speedup vs baseline: 1.7074x; 1.7074x over previous
import jax
import jax.numpy as jnp
from jax.experimental import pallas as pl

_B, _D, _H, _W, _K = 16, 64, 24, 24, 1024
_HW = _H * _W


def _vq_kernel(x_ref, wt_ref, w_ref, out_ref, idx_ref):
    x = x_ref[0]
    wt = wt_ref[...]
    w = w_ref[...]
    scores = jax.lax.dot_general(
        wt, x, (((1,), (0,)), ((), ())),
        preferred_element_type=jnp.float32)
    x_sq = jnp.sum(x * x, axis=0, keepdims=True)
    w_sq = jnp.sum(wt * wt, axis=1, keepdims=True)
    d2 = jnp.maximum(w_sq + x_sq - 2.0 * scores, 0.0)
    idx = jnp.argmin(d2, axis=0).astype(jnp.int32)
    onehot = (jax.lax.broadcasted_iota(jnp.int32, (_K, _HW), 0)
              == idx[None, :]).astype(jnp.float32)
    res = jax.lax.dot_general(
        w, onehot, (((1,), (0,)), ((), ())),
        preferred_element_type=jnp.float32)
    out_ref[0] = res
    idx_ref[0, 0] = idx


def kernel(x, weight):
    x3 = x.reshape(_B, _D, _HW)
    wt = weight.T
    result, idx = pl.pallas_call(
        _vq_kernel,
        grid=(_B,),
        in_specs=[
            pl.BlockSpec((1, _D, _HW), lambda b: (b, 0, 0)),
            pl.BlockSpec((_K, _D), lambda b: (0, 0)),
            pl.BlockSpec((_D, _K), lambda b: (0, 0)),
        ],
        out_specs=[
            pl.BlockSpec((1, _D, _HW), lambda b: (b, 0, 0)),
            pl.BlockSpec((1, 1, _HW), lambda b: (b, 0, 0)),
        ],
        out_shape=[
            jax.ShapeDtypeStruct((_B, _D, _HW), jnp.float32),
            jax.ShapeDtypeStruct((_B, 1, _HW), jnp.int32),
        ],
    )(x3, wt, weight)
    return result.reshape(_B, _D, _H, _W), idx.reshape(_B, _H, _W)


# speedup vs baseline: 1.8372x; 1.0761x over previous
import jax
import jax.numpy as jnp
from jax.experimental import pallas as pl

_B, _D, _H, _W, _K = 16, 64, 24, 24, 1024
_HW = _H * _W


def _vq_kernel(x_ref, wt_ref, out_ref, idx_ref):
    x = x_ref[0]
    wt = wt_ref[...]
    scores = jax.lax.dot_general(
        wt, x, (((1,), (0,)), ((), ())),
        preferred_element_type=jnp.float32)
    w_sq = jnp.sum(wt * wt, axis=1, keepdims=True)
    d2 = w_sq - 2.0 * scores
    idx = jnp.argmin(d2, axis=0).astype(jnp.int32)
    onehot = (jax.lax.broadcasted_iota(jnp.int32, (_K, _HW), 0)
              == idx[None, :]).astype(jnp.float32)
    res = jax.lax.dot_general(
        wt, onehot, (((0,), (0,)), ((), ())),
        preferred_element_type=jnp.float32)
    out_ref[0] = res
    idx_ref[0, 0] = idx


def kernel(x, weight):
    x3 = x.reshape(_B, _D, _HW)
    wt = weight.T
    result, idx = pl.pallas_call(
        _vq_kernel,
        grid=(_B,),
        in_specs=[
            pl.BlockSpec((1, _D, _HW), lambda b: (b, 0, 0)),
            pl.BlockSpec((_K, _D), lambda b: (0, 0)),
        ],
        out_specs=[
            pl.BlockSpec((1, _D, _HW), lambda b: (b, 0, 0)),
            pl.BlockSpec((1, 1, _HW), lambda b: (b, 0, 0)),
        ],
        out_shape=[
            jax.ShapeDtypeStruct((_B, _D, _HW), jnp.float32),
            jax.ShapeDtypeStruct((_B, 1, _HW), jnp.int32),
        ],
    )(x3, wt)
    return result.reshape(_B, _D, _H, _W), idx.reshape(_B, _H, _W)


# speedup vs baseline: 1.8591x; 1.0119x over previous
import jax
import jax.numpy as jnp
from jax.experimental import pallas as pl

_B, _D, _H, _W, _K = 16, 64, 24, 24, 1024
_HW = _H * _W


def _tree_argmin(v):
    k = v.shape[0]
    rel = None
    while k > 1:
        h = k // 2
        vlo, vhi = v[:h], v[h:]
        take = vhi < vlo
        v = jnp.where(take, vhi, vlo)
        if rel is None:
            rel = jnp.where(take, jnp.int32(h), jnp.int32(0))
        else:
            rel = jnp.where(take, rel[h:] + jnp.int32(h), rel[:h])
        k = h
    return rel


def _vq_kernel(x_ref, w_ref, out_ref, idx_ref):
    x = x_ref[0]
    w = w_ref[...]
    w_sq = jnp.sum(w * w, axis=0, keepdims=True)
    lhs = jnp.concatenate([w, w_sq], axis=0)
    rhs = jnp.concatenate(
        [-2.0 * x, jnp.ones((1, _HW), jnp.float32)], axis=0)
    d2 = jax.lax.dot_general(
        lhs, rhs, (((0,), (0,)), ((), ())),
        preferred_element_type=jnp.float32)
    idx = _tree_argmin(d2)
    onehot = (jax.lax.broadcasted_iota(jnp.int32, (_K, _HW), 0)
              == idx).astype(jnp.float32)
    res = jax.lax.dot_general(
        w, onehot, (((1,), (0,)), ((), ())),
        preferred_element_type=jnp.float32)
    out_ref[0] = res
    idx_ref[0, 0] = idx[0]


def kernel(x, weight):
    x3 = x.reshape(_B, _D, _HW)
    result, idx = pl.pallas_call(
        _vq_kernel,
        grid=(_B,),
        in_specs=[
            pl.BlockSpec((1, _D, _HW), lambda b: (b, 0, 0)),
            pl.BlockSpec((_D, _K), lambda b: (0, 0)),
        ],
        out_specs=[
            pl.BlockSpec((1, _D, _HW), lambda b: (b, 0, 0)),
            pl.BlockSpec((1, 1, _HW), lambda b: (b, 0, 0)),
        ],
        out_shape=[
            jax.ShapeDtypeStruct((_B, _D, _HW), jnp.float32),
            jax.ShapeDtypeStruct((_B, 1, _HW), jnp.int32),
        ],
    )(x3, weight)
    return result.reshape(_B, _D, _H, _W), idx.reshape(_B, _H, _W)
